# Initial kernel scaffold; baseline (speedup 1.0000x reference)
import jax
import jax.numpy as jnp
from jax.experimental import pallas as pl


def kernel(x, batch, W, b):
    raise NotImplementedError("write your pallas kernel here")


# speedup vs baseline: 3.3574x; 3.3574x over previous
import functools

import jax
import jax.numpy as jnp
from jax import lax
from jax.experimental import pallas as pl
from jax.experimental.pallas import tpu as pltpu
from jax.experimental.pallas import tpu_sc as plsc

NROWS = 100000
D = 128
S = 2048
CNT_W = 128

NC = 2
NS = 16
NW = NC * NS

ACTIVE_W = 25
ROWS_PER_W = 4000
CHUNK = 80
NCHUNK = ROWS_PER_W // CHUNK

_mesh = plsc.VectorSubcoreMesh(core_axis_name="c", subcore_axis_name="s")


@functools.partial(
    pl.kernel,
    out_type=[
        jax.ShapeDtypeStruct((NC, S, D), jnp.float32),
        jax.ShapeDtypeStruct((NC, S, CNT_W), jnp.float32),
    ],
    mesh=_mesh,
    scratch_types=[
        pltpu.VMEM_SHARED((S, D), jnp.float32),
        pltpu.VMEM_SHARED((S, CNT_W), jnp.float32),
        pltpu.VMEM((CHUNK,), jnp.int32),
        pltpu.VMEM((CHUNK, D), jnp.float32),
        pltpu.VMEM((CHUNK, CNT_W), jnp.float32),
    ],
)
def _sc_segment_sums(x_hbm, batch_hbm, zsum_hbm, zcnt_hbm, ones_hbm,
                     sums_out, cnts_out,
                     acc_sum, acc_cnt, idx_v, x_v, ones_v):
    cid = lax.axis_index("c")
    sid = lax.axis_index("s")
    wid = sid * NC + cid

    pltpu.sync_copy(ones_hbm, ones_v)

    seg0 = sid * (S // NS)
    pltpu.sync_copy(zsum_hbm.at[pl.ds(seg0, S // NS)],
                    acc_sum.at[pl.ds(seg0, S // NS)])
    pltpu.sync_copy(zcnt_hbm.at[pl.ds(seg0, S // NS)],
                    acc_cnt.at[pl.ds(seg0, S // NS)])
    plsc.subcore_barrier()

    @pl.when(wid < ACTIVE_W)
    def _():
        def body(i, carry):
            base = pl.multiple_of(wid * ROWS_PER_W + i * CHUNK, 8)
            pltpu.sync_copy(batch_hbm.at[pl.ds(base, CHUNK)], idx_v)
            pltpu.sync_copy(x_hbm.at[pl.ds(base, CHUNK), :], x_v)
            pltpu.sync_copy(x_v, acc_sum.at[idx_v], add=True)
            pltpu.sync_copy(ones_v, acc_cnt.at[idx_v], add=True)
            return carry

        lax.fori_loop(0, NCHUNK, body, 0)

    plsc.subcore_barrier()
    pltpu.sync_copy(acc_sum.at[pl.ds(seg0, S // NS)],
                    sums_out.at[cid, pl.ds(seg0, S // NS)])
    pltpu.sync_copy(acc_cnt.at[pl.ds(seg0, S // NS)],
                    cnts_out.at[cid, pl.ds(seg0, S // NS)])


def _tc_body(s_ref, c_ref, w_ref, b_ref, o_ref):
    sums = s_ref[0] + s_ref[1]
    cnt = c_ref[0, :, 0] + c_ref[1, :, 0]
    pooled = sums / jnp.maximum(cnt, 1.0)[:, None]
    o_ref[...] = (
        jnp.dot(pooled, w_ref[...], preferred_element_type=jnp.float32)
        + b_ref[...]
    )


def kernel(x, batch, W, b):
    batch = batch.astype(jnp.int32)
    zsum = jnp.zeros((S, D), jnp.float32)
    zcnt = jnp.zeros((S, CNT_W), jnp.float32)
    ones = jnp.ones((CHUNK, CNT_W), jnp.float32)
    sums2, cnts2 = _sc_segment_sums(x, batch, zsum, zcnt, ones)
    out = pl.pallas_call(
        _tc_body,
        out_shape=jax.ShapeDtypeStruct((S, D), jnp.float32),
    )(sums2, cnts2, W, b.reshape(1, D))
    return out


# speedup vs baseline: 5.9605x; 1.7753x over previous
import functools

import jax
import jax.numpy as jnp
from jax import lax
from jax.experimental import pallas as pl
from jax.experimental.pallas import tpu as pltpu
from jax.experimental.pallas import tpu_sc as plsc

NROWS = 100000
D = 128
S = 2048
CNT_W = 128

NC = 2
NS = 16
NW = NC * NS

CHUNK = 128
BASE_CHUNKS = 24
EXTRA_CHUNKS = (NROWS // CHUNK) - NW * BASE_CHUNKS
TAIL_BASE = (NROWS // CHUNK) * CHUNK
TAIL = NROWS - TAIL_BASE

_mesh = plsc.VectorSubcoreMesh(core_axis_name="c", subcore_axis_name="s")


@functools.partial(
    pl.kernel,
    out_type=[
        jax.ShapeDtypeStruct((NC, S, D), jnp.float32),
        jax.ShapeDtypeStruct((NC, S, CNT_W), jnp.float32),
    ],
    mesh=_mesh,
    scratch_types=[
        pltpu.VMEM_SHARED((S, D), jnp.float32),
        pltpu.VMEM_SHARED((S, CNT_W), jnp.float32),
        pltpu.VMEM((CHUNK,), jnp.int32),
        pltpu.VMEM((CHUNK,), jnp.int32),
        pltpu.VMEM((CHUNK, D), jnp.float32),
        pltpu.VMEM((CHUNK, D), jnp.float32),
        pltpu.VMEM((CHUNK, CNT_W), jnp.float32),
        pltpu.VMEM((TAIL,), jnp.int32),
        pltpu.SemaphoreType.DMA,
        pltpu.SemaphoreType.DMA,
    ],
)
def _sc_segment_sums(x_hbm, batch_hbm, zsum_hbm, zcnt_hbm, ones_hbm,
                     sums_out, cnts_out,
                     acc_sum, acc_cnt, idx_a, idx_b, x_a, x_b, ones_v,
                     idx_t, sem_a, sem_b):
    cid = lax.axis_index("c")
    sid = lax.axis_index("s")
    wid = sid * NC + cid

    pltpu.sync_copy(ones_hbm, ones_v)

    seg0 = sid * (S // NS)
    pltpu.sync_copy(zsum_hbm.at[pl.ds(seg0, S // NS)],
                    acc_sum.at[pl.ds(seg0, S // NS)])
    pltpu.sync_copy(zcnt_hbm.at[pl.ds(seg0, S // NS)],
                    acc_cnt.at[pl.ds(seg0, S // NS)])
    plsc.subcore_barrier()

    def start_load(g, idx_ref, x_ref, sem):
        base = pl.multiple_of(g * CHUNK, 8)
        pltpu.async_copy(batch_hbm.at[pl.ds(base, CHUNK)], idx_ref, sem)
        pltpu.async_copy(x_hbm.at[pl.ds(base, CHUNK), :], x_ref, sem)

    def wait_load(idx_ref, x_ref, sem):
        pltpu.make_async_copy(batch_hbm.at[pl.ds(0, CHUNK)], idx_ref, sem).wait()
        pltpu.make_async_copy(x_hbm.at[pl.ds(0, CHUNK), :], x_ref, sem).wait()

    def scatter(idx_ref, x_ref):
        pltpu.sync_copy(x_ref, acc_sum.at[idx_ref], add=True)
        pltpu.sync_copy(ones_v, acc_cnt.at[idx_ref], add=True)

    lo = wid * BASE_CHUNKS
    start_load(lo, idx_a, x_a, sem_a)

    def body(k, carry):
        g = lo + k

        def step(cur_idx, cur_x, cur_sem, nxt_idx, nxt_x, nxt_sem):
            @pl.when(k + 1 < BASE_CHUNKS)
            def _():
                start_load(g + 1, nxt_idx, nxt_x, nxt_sem)

            wait_load(cur_idx, cur_x, cur_sem)
            scatter(cur_idx, cur_x)

        @pl.when(k % 2 == 0)
        def _():
            step(idx_a, x_a, sem_a, idx_b, x_b, sem_b)

        @pl.when(k % 2 == 1)
        def _():
            step(idx_b, x_b, sem_b, idx_a, x_a, sem_a)

        return carry

    lax.fori_loop(0, BASE_CHUNKS, body, 0)

    @pl.when(wid < EXTRA_CHUNKS)
    def _():
        g = NW * BASE_CHUNKS + wid
        start_load(g, idx_a, x_a, sem_a)
        wait_load(idx_a, x_a, sem_a)
        scatter(idx_a, x_a)

    @pl.when(wid == NW - 1)
    def _():
        pltpu.sync_copy(batch_hbm.at[pl.ds(TAIL_BASE, TAIL)], idx_t)
        pltpu.sync_copy(x_hbm.at[pl.ds(TAIL_BASE, TAIL), :],
                        x_a.at[pl.ds(0, TAIL), :])
        pltpu.sync_copy(x_a.at[pl.ds(0, TAIL), :],
                        acc_sum.at[idx_t], add=True)
        pltpu.sync_copy(ones_v.at[pl.ds(0, TAIL), :],
                        acc_cnt.at[idx_t], add=True)

    plsc.subcore_barrier()
    pltpu.sync_copy(acc_sum.at[pl.ds(seg0, S // NS)],
                    sums_out.at[cid, pl.ds(seg0, S // NS)])
    pltpu.sync_copy(acc_cnt.at[pl.ds(seg0, S // NS)],
                    cnts_out.at[cid, pl.ds(seg0, S // NS)])


def _tc_body(s_ref, c_ref, w_ref, b_ref, o_ref):
    sums = s_ref[0] + s_ref[1]
    cnt = c_ref[0, :, 0] + c_ref[1, :, 0]
    pooled = sums / jnp.maximum(cnt, 1.0)[:, None]
    o_ref[...] = (
        jnp.dot(pooled, w_ref[...], preferred_element_type=jnp.float32)
        + b_ref[...]
    )


def kernel(x, batch, W, b):
    batch = batch.astype(jnp.int32)
    zsum = jnp.zeros((S, D), jnp.float32)
    zcnt = jnp.zeros((S, CNT_W), jnp.float32)
    ones = jnp.ones((CHUNK, CNT_W), jnp.float32)
    sums2, cnts2 = _sc_segment_sums(x, batch, zsum, zcnt, ones)
    out = pl.pallas_call(
        _tc_body,
        out_shape=jax.ShapeDtypeStruct((S, D), jnp.float32),
    )(sums2, cnts2, W, b.reshape(1, D))
    return out


# speedup vs baseline: 6.0093x; 1.0082x over previous
import functools

import jax
import jax.numpy as jnp
from jax import lax
from jax.experimental import pallas as pl
from jax.experimental.pallas import tpu as pltpu
from jax.experimental.pallas import tpu_sc as plsc

NROWS = 100000
D = 128
S = 2048
CNT_W = 128

NC = 2
NS = 16
NW = NC * NS

CHUNK = 128
BASE_CHUNKS = 24
EXTRA_CHUNKS = (NROWS // CHUNK) - NW * BASE_CHUNKS
TAIL_BASE = (NROWS // CHUNK) * CHUNK
TAIL = NROWS - TAIL_BASE

_mesh = plsc.VectorSubcoreMesh(core_axis_name="c", subcore_axis_name="s")


@functools.partial(
    pl.kernel,
    out_type=[
        jax.ShapeDtypeStruct((NC, S, D), jnp.float32),
        jax.ShapeDtypeStruct((NC, S, CNT_W), jnp.float32),
    ],
    mesh=_mesh,
    scratch_types=[
        pltpu.VMEM_SHARED((S, D), jnp.float32),
        pltpu.VMEM_SHARED((S, CNT_W), jnp.float32),
        pltpu.VMEM((CHUNK,), jnp.int32),
        pltpu.VMEM((CHUNK,), jnp.int32),
        pltpu.VMEM((CHUNK, D), jnp.float32),
        pltpu.VMEM((CHUNK, D), jnp.float32),
        pltpu.VMEM((CHUNK, CNT_W), jnp.float32),
        pltpu.VMEM((TAIL,), jnp.int32),
        pltpu.SemaphoreType.DMA,
        pltpu.SemaphoreType.DMA,
        pltpu.SemaphoreType.DMA,
        pltpu.SemaphoreType.DMA,
    ],
)
def _sc_segment_sums(x_hbm, batch_hbm, zsum_hbm, zcnt_hbm, ones_hbm,
                     sums_out, cnts_out,
                     acc_sum, acc_cnt, idx_a, idx_b, x_a, x_b, ones_v,
                     idx_t, sem_a, sem_b, ssem_a, ssem_b):
    cid = lax.axis_index("c")
    sid = lax.axis_index("s")
    wid = sid * NC + cid

    pltpu.sync_copy(ones_hbm, ones_v)

    seg0 = sid * (S // NS)
    pltpu.sync_copy(zsum_hbm.at[pl.ds(seg0, S // NS)],
                    acc_sum.at[pl.ds(seg0, S // NS)])
    pltpu.sync_copy(zcnt_hbm.at[pl.ds(seg0, S // NS)],
                    acc_cnt.at[pl.ds(seg0, S // NS)])
    plsc.subcore_barrier()

    def start_load(g, idx_ref, x_ref, sem):
        base = pl.multiple_of(g * CHUNK, 8)
        pltpu.async_copy(batch_hbm.at[pl.ds(base, CHUNK)], idx_ref, sem)
        pltpu.async_copy(x_hbm.at[pl.ds(base, CHUNK), :], x_ref, sem)

    def wait_load(idx_ref, x_ref, sem):
        pltpu.make_async_copy(batch_hbm.at[pl.ds(0, CHUNK)], idx_ref, sem).wait()
        pltpu.make_async_copy(x_hbm.at[pl.ds(0, CHUNK), :], x_ref, sem).wait()

    def start_scatter(idx_ref, x_ref, ssem):
        pltpu.async_copy(x_ref, acc_sum.at[idx_ref], ssem, add=True)
        pltpu.async_copy(ones_v, acc_cnt.at[idx_ref], ssem, add=True)

    def wait_scatter(idx_ref, x_ref, ssem):
        pltpu.make_async_copy(x_ref, acc_sum.at[idx_ref], ssem).wait()
        pltpu.make_async_copy(ones_v, acc_cnt.at[idx_ref], ssem).wait()

    lo = wid * BASE_CHUNKS
    start_load(lo, idx_a, x_a, sem_a)

    def body(k, carry):
        g = lo + k

        def step(cur_idx, cur_x, cur_sem, cur_ssem, nxt_idx, nxt_x, nxt_sem,
                 nxt_ssem):
            @pl.when(k + 1 < BASE_CHUNKS)
            def _():
                @pl.when(k >= 1)
                def _():
                    wait_scatter(nxt_idx, nxt_x, nxt_ssem)

                start_load(g + 1, nxt_idx, nxt_x, nxt_sem)

            wait_load(cur_idx, cur_x, cur_sem)
            start_scatter(cur_idx, cur_x, cur_ssem)

        @pl.when(k % 2 == 0)
        def _():
            step(idx_a, x_a, sem_a, ssem_a, idx_b, x_b, sem_b, ssem_b)

        @pl.when(k % 2 == 1)
        def _():
            step(idx_b, x_b, sem_b, ssem_b, idx_a, x_a, sem_a, ssem_a)

        return carry

    lax.fori_loop(0, BASE_CHUNKS, body, 0)

    wait_scatter(idx_a, x_a, ssem_a)
    wait_scatter(idx_b, x_b, ssem_b)

    @pl.when(wid < EXTRA_CHUNKS)
    def _():
        g = NW * BASE_CHUNKS + wid
        start_load(g, idx_a, x_a, sem_a)
        wait_load(idx_a, x_a, sem_a)
        start_scatter(idx_a, x_a, ssem_a)
        wait_scatter(idx_a, x_a, ssem_a)

    @pl.when(wid == NW - 1)
    def _():
        pltpu.sync_copy(batch_hbm.at[pl.ds(TAIL_BASE, TAIL)], idx_t)
        pltpu.sync_copy(x_hbm.at[pl.ds(TAIL_BASE, TAIL), :],
                        x_a.at[pl.ds(0, TAIL), :])
        pltpu.sync_copy(x_a.at[pl.ds(0, TAIL), :],
                        acc_sum.at[idx_t], add=True)
        pltpu.sync_copy(ones_v.at[pl.ds(0, TAIL), :],
                        acc_cnt.at[idx_t], add=True)

    plsc.subcore_barrier()
    pltpu.sync_copy(acc_sum.at[pl.ds(seg0, S // NS)],
                    sums_out.at[cid, pl.ds(seg0, S // NS)])
    pltpu.sync_copy(acc_cnt.at[pl.ds(seg0, S // NS)],
                    cnts_out.at[cid, pl.ds(seg0, S // NS)])


def _tc_body(s_ref, c_ref, w_ref, b_ref, o_ref):
    sums = s_ref[0] + s_ref[1]
    cnt = c_ref[0, :, 0] + c_ref[1, :, 0]
    pooled = sums / jnp.maximum(cnt, 1.0)[:, None]
    o_ref[...] = (
        jnp.dot(pooled, w_ref[...], preferred_element_type=jnp.float32)
        + b_ref[...]
    )


def kernel(x, batch, W, b):
    batch = batch.astype(jnp.int32)
    zsum = jnp.zeros((S, D), jnp.float32)
    zcnt = jnp.zeros((S, CNT_W), jnp.float32)
    ones = jnp.ones((CHUNK, CNT_W), jnp.float32)
    sums2, cnts2 = _sc_segment_sums(x, batch, zsum, zcnt, ones)
    out = pl.pallas_call(
        _tc_body,
        out_shape=jax.ShapeDtypeStruct((S, D), jnp.float32),
    )(sums2, cnts2, W, b.reshape(1, D))
    return out


# speedup vs baseline: 8.3616x; 1.3914x over previous
import functools

import jax
import jax.numpy as jnp
from jax import lax
from jax.experimental import pallas as pl
from jax.experimental.pallas import tpu as pltpu
from jax.experimental.pallas import tpu_sc as plsc

NROWS = 100000
D = 128
S = 2048

NC = 2
NS = 16
NW = NC * NS

CHUNK = 128
BASE_CHUNKS = 24
EXTRA_CHUNKS = (NROWS // CHUNK) - NW * BASE_CHUNKS
TAIL_BASE = (NROWS // CHUNK) * CHUNK
TAIL = NROWS - TAIL_BASE

HIST_BLOCKS = 8
HIST_B = 12544
HIST_PAD = HIST_BLOCKS * HIST_B - NROWS

_mesh = plsc.VectorSubcoreMesh(core_axis_name="c", subcore_axis_name="s")


@functools.partial(
    pl.kernel,
    out_type=jax.ShapeDtypeStruct((NC, S, D), jnp.float32),
    mesh=_mesh,
    scratch_types=[
        pltpu.VMEM_SHARED((S, D), jnp.float32),
        pltpu.VMEM((CHUNK,), jnp.int32),
        pltpu.VMEM((CHUNK,), jnp.int32),
        pltpu.VMEM((CHUNK, D), jnp.float32),
        pltpu.VMEM((CHUNK, D), jnp.float32),
        pltpu.VMEM((TAIL,), jnp.int32),
        pltpu.SemaphoreType.DMA,
        pltpu.SemaphoreType.DMA,
        pltpu.SemaphoreType.DMA,
        pltpu.SemaphoreType.DMA,
    ],
)
def _sc_segment_sums(x_hbm, batch_hbm, zsum_hbm,
                     sums_out,
                     acc_sum, idx_a, idx_b, x_a, x_b,
                     idx_t, sem_a, sem_b, ssem_a, ssem_b):
    cid = lax.axis_index("c")
    sid = lax.axis_index("s")
    wid = sid * NC + cid

    seg0 = sid * (S // NS)
    pltpu.sync_copy(zsum_hbm.at[pl.ds(seg0, S // NS)],
                    acc_sum.at[pl.ds(seg0, S // NS)])
    plsc.subcore_barrier()

    def start_load(g, idx_ref, x_ref, sem):
        base = pl.multiple_of(g * CHUNK, 8)
        pltpu.async_copy(batch_hbm.at[pl.ds(base, CHUNK)], idx_ref, sem)
        pltpu.async_copy(x_hbm.at[pl.ds(base, CHUNK), :], x_ref, sem)

    def wait_load(idx_ref, x_ref, sem):
        pltpu.make_async_copy(batch_hbm.at[pl.ds(0, CHUNK)], idx_ref, sem).wait()
        pltpu.make_async_copy(x_hbm.at[pl.ds(0, CHUNK), :], x_ref, sem).wait()

    def start_scatter(idx_ref, x_ref, ssem):
        pltpu.async_copy(x_ref, acc_sum.at[idx_ref], ssem, add=True)

    def wait_scatter(idx_ref, x_ref, ssem):
        pltpu.make_async_copy(x_ref, acc_sum.at[idx_ref], ssem).wait()

    lo = wid * BASE_CHUNKS
    start_load(lo, idx_a, x_a, sem_a)

    def body(k, carry):
        g = lo + k

        def step(cur_idx, cur_x, cur_sem, cur_ssem, nxt_idx, nxt_x, nxt_sem,
                 nxt_ssem):
            @pl.when(k + 1 < BASE_CHUNKS)
            def _():
                @pl.when(k >= 1)
                def _():
                    wait_scatter(nxt_idx, nxt_x, nxt_ssem)

                start_load(g + 1, nxt_idx, nxt_x, nxt_sem)

            wait_load(cur_idx, cur_x, cur_sem)
            start_scatter(cur_idx, cur_x, cur_ssem)

        @pl.when(k % 2 == 0)
        def _():
            step(idx_a, x_a, sem_a, ssem_a, idx_b, x_b, sem_b, ssem_b)

        @pl.when(k % 2 == 1)
        def _():
            step(idx_b, x_b, sem_b, ssem_b, idx_a, x_a, sem_a, ssem_a)

        return carry

    lax.fori_loop(0, BASE_CHUNKS, body, 0)

    wait_scatter(idx_a, x_a, ssem_a)
    wait_scatter(idx_b, x_b, ssem_b)

    @pl.when(wid < EXTRA_CHUNKS)
    def _():
        g = NW * BASE_CHUNKS + wid
        start_load(g, idx_a, x_a, sem_a)
        wait_load(idx_a, x_a, sem_a)
        start_scatter(idx_a, x_a, ssem_a)
        wait_scatter(idx_a, x_a, ssem_a)

    @pl.when(wid == NW - 1)
    def _():
        pltpu.sync_copy(batch_hbm.at[pl.ds(TAIL_BASE, TAIL)], idx_t)
        pltpu.sync_copy(x_hbm.at[pl.ds(TAIL_BASE, TAIL), :],
                        x_a.at[pl.ds(0, TAIL), :])
        pltpu.sync_copy(x_a.at[pl.ds(0, TAIL), :],
                        acc_sum.at[idx_t], add=True)

    plsc.subcore_barrier()
    pltpu.sync_copy(acc_sum.at[pl.ds(seg0, S // NS)],
                    sums_out.at[cid, pl.ds(seg0, S // NS)])


def _tc_hist_body(ids_ref, o_ref):
    i = pl.program_id(0)
    ids = ids_ref[0, 0, :]
    hh = ids >> 7
    ll = ids & 127
    hoh = (hh[:, None] == lax.broadcasted_iota(jnp.int32, (HIST_B, 16), 1))
    loh = (ll[:, None] == lax.broadcasted_iota(jnp.int32, (HIST_B, 128), 1))
    partial = lax.dot_general(
        hoh.astype(jnp.bfloat16), loh.astype(jnp.bfloat16),
        dimension_numbers=(((0,), (0,)), ((), ())),
        preferred_element_type=jnp.float32,
    )

    @pl.when(i == 0)
    def _():
        o_ref[...] = partial

    @pl.when(i > 0)
    def _():
        o_ref[...] += partial


def _tc_finish_body(s_ref, c_ref, w_ref, b_ref, o_ref):
    sums = s_ref[0] + s_ref[1]
    pooled = sums / jnp.maximum(c_ref[...], 1.0)
    o_ref[...] = (
        jnp.dot(pooled, w_ref[...], preferred_element_type=jnp.float32)
        + b_ref[...]
    )


def kernel(x, batch, W, b):
    batch = batch.astype(jnp.int32)
    zsum = jnp.zeros((S, D), jnp.float32)
    sums2 = _sc_segment_sums(x, batch, zsum)

    ids_padded = jnp.concatenate(
        [batch, jnp.full((HIST_PAD,), -1, jnp.int32)]
    ).reshape(HIST_BLOCKS, 1, HIST_B)
    hist = pl.pallas_call(
        _tc_hist_body,
        grid=(HIST_BLOCKS,),
        in_specs=[pl.BlockSpec((1, 1, HIST_B), lambda i: (i, 0, 0))],
        out_specs=pl.BlockSpec((16, 128), lambda i: (0, 0)),
        out_shape=jax.ShapeDtypeStruct((16, 128), jnp.float32),
    )(ids_padded)
    counts = hist.reshape(S, 1)

    out = pl.pallas_call(
        _tc_finish_body,
        out_shape=jax.ShapeDtypeStruct((S, D), jnp.float32),
    )(sums2, counts, W, b.reshape(1, D))
    return out


# speedup vs baseline: 8.3847x; 1.0028x over previous
import functools

import jax
import jax.numpy as jnp
from jax import lax
from jax.experimental import pallas as pl
from jax.experimental.pallas import tpu as pltpu
from jax.experimental.pallas import tpu_sc as plsc

NROWS = 100000
D = 128
S = 2048

NC = 2
NS = 16
NW = NC * NS

CHUNK = 128
BASE_CHUNKS = 24
EXTRA_CHUNKS = (NROWS // CHUNK) - NW * BASE_CHUNKS
TAIL_BASE = (NROWS // CHUNK) * CHUNK
TAIL = NROWS - TAIL_BASE

HIST_BLOCKS = 8
HIST_B = 12544
HIST_PAD = HIST_BLOCKS * HIST_B - NROWS

_mesh = plsc.VectorSubcoreMesh(core_axis_name="c", subcore_axis_name="s")


@functools.partial(
    pl.kernel,
    out_type=jax.ShapeDtypeStruct((NC, S, D), jnp.float32),
    mesh=_mesh,
    scratch_types=[
        pltpu.VMEM_SHARED((S, D), jnp.float32),
        pltpu.VMEM((CHUNK,), jnp.int32),
        pltpu.VMEM((CHUNK,), jnp.int32),
        pltpu.VMEM((CHUNK, D), jnp.float32),
        pltpu.VMEM((CHUNK, D), jnp.float32),
        pltpu.VMEM((TAIL,), jnp.int32),
        pltpu.VMEM((TAIL, D), jnp.float32),
        pltpu.SemaphoreType.DMA,
        pltpu.SemaphoreType.DMA,
        pltpu.SemaphoreType.DMA,
        pltpu.SemaphoreType.DMA,
        pltpu.SemaphoreType.DMA,
    ],
)
def _sc_segment_sums(x_hbm, batch_hbm, zsum_hbm,
                     sums_out,
                     acc_sum, idx_a, idx_b, x_a, x_b,
                     idx_t, x_t, sem_a, sem_b, ssem_a, ssem_b, sem_t):
    cid = lax.axis_index("c")
    sid = lax.axis_index("s")
    wid = sid * NC + cid

    def start_load(g, idx_ref, x_ref, sem):
        base = pl.multiple_of(g * CHUNK, 8)
        pltpu.async_copy(batch_hbm.at[pl.ds(base, CHUNK)], idx_ref, sem)
        pltpu.async_copy(x_hbm.at[pl.ds(base, CHUNK), :], x_ref, sem)

    def wait_load(idx_ref, x_ref, sem):
        pltpu.make_async_copy(batch_hbm.at[pl.ds(0, CHUNK)], idx_ref, sem).wait()
        pltpu.make_async_copy(x_hbm.at[pl.ds(0, CHUNK), :], x_ref, sem).wait()

    def start_scatter(idx_ref, x_ref, ssem):
        pltpu.async_copy(x_ref, acc_sum.at[idx_ref], ssem, add=True)

    def wait_scatter(idx_ref, x_ref, ssem):
        pltpu.make_async_copy(x_ref, acc_sum.at[idx_ref], ssem).wait()

    lo = wid * BASE_CHUNKS
    start_load(lo, idx_a, x_a, sem_a)

    @pl.when(wid == NW - 1)
    def _():
        pltpu.async_copy(batch_hbm.at[pl.ds(TAIL_BASE, TAIL)], idx_t, sem_t)
        pltpu.async_copy(x_hbm.at[pl.ds(TAIL_BASE, TAIL), :], x_t, sem_t)

    seg0 = sid * (S // NS)
    pltpu.sync_copy(zsum_hbm.at[pl.ds(seg0, S // NS)],
                    acc_sum.at[pl.ds(seg0, S // NS)])
    plsc.subcore_barrier()

    my_chunks = BASE_CHUNKS + jnp.where(wid < EXTRA_CHUNKS, 1, 0)

    def chunk_id(k):
        return jnp.where(k < BASE_CHUNKS, lo + k, NW * BASE_CHUNKS + wid)

    def body(k, carry):
        def step(cur_idx, cur_x, cur_sem, cur_ssem, nxt_idx, nxt_x, nxt_sem,
                 nxt_ssem):
            @pl.when(k + 1 < my_chunks)
            def _():
                @pl.when(k >= 1)
                def _():
                    wait_scatter(nxt_idx, nxt_x, nxt_ssem)

                start_load(chunk_id(k + 1), nxt_idx, nxt_x, nxt_sem)

            wait_load(cur_idx, cur_x, cur_sem)
            start_scatter(cur_idx, cur_x, cur_ssem)

        @pl.when(k % 2 == 0)
        def _():
            step(idx_a, x_a, sem_a, ssem_a, idx_b, x_b, sem_b, ssem_b)

        @pl.when(k % 2 == 1)
        def _():
            step(idx_b, x_b, sem_b, ssem_b, idx_a, x_a, sem_a, ssem_a)

        return carry

    lax.fori_loop(0, my_chunks, body, 0)

    wait_scatter(idx_a, x_a, ssem_a)
    wait_scatter(idx_b, x_b, ssem_b)

    @pl.when(wid == NW - 1)
    def _():
        pltpu.make_async_copy(batch_hbm.at[pl.ds(0, TAIL)], idx_t, sem_t).wait()
        pltpu.make_async_copy(x_hbm.at[pl.ds(0, TAIL), :], x_t, sem_t).wait()
        pltpu.sync_copy(x_t, acc_sum.at[idx_t], add=True)

    plsc.subcore_barrier()
    pltpu.sync_copy(acc_sum.at[pl.ds(seg0, S // NS)],
                    sums_out.at[cid, pl.ds(seg0, S // NS)])


def _tc_hist_body(ids_ref, o_ref):
    i = pl.program_id(0)
    ids = ids_ref[0, 0, :]
    hh = ids >> 7
    ll = ids & 127
    hoh = (hh[:, None] == lax.broadcasted_iota(jnp.int32, (HIST_B, 16), 1))
    loh = (ll[:, None] == lax.broadcasted_iota(jnp.int32, (HIST_B, 128), 1))
    partial = lax.dot_general(
        hoh.astype(jnp.bfloat16), loh.astype(jnp.bfloat16),
        dimension_numbers=(((0,), (0,)), ((), ())),
        preferred_element_type=jnp.float32,
    )

    @pl.when(i == 0)
    def _():
        o_ref[...] = partial

    @pl.when(i > 0)
    def _():
        o_ref[...] += partial


def _tc_finish_body(s_ref, c_ref, w_ref, b_ref, o_ref):
    sums = s_ref[0] + s_ref[1]
    pooled = sums / jnp.maximum(c_ref[...], 1.0)
    o_ref[...] = (
        jnp.dot(pooled, w_ref[...], preferred_element_type=jnp.float32)
        + b_ref[...]
    )


def kernel(x, batch, W, b):
    batch = batch.astype(jnp.int32)
    zsum = jnp.zeros((S, D), jnp.float32)
    sums2 = _sc_segment_sums(x, batch, zsum)

    ids_padded = jnp.concatenate(
        [batch, jnp.full((HIST_PAD,), -1, jnp.int32)]
    ).reshape(HIST_BLOCKS, 1, HIST_B)
    hist = pl.pallas_call(
        _tc_hist_body,
        grid=(HIST_BLOCKS,),
        in_specs=[pl.BlockSpec((1, 1, HIST_B), lambda i: (i, 0, 0))],
        out_specs=pl.BlockSpec((16, 128), lambda i: (0, 0)),
        out_shape=jax.ShapeDtypeStruct((16, 128), jnp.float32),
    )(ids_padded)
    counts = hist.reshape(S, 1)

    out = pl.pallas_call(
        _tc_finish_body,
        out_shape=jax.ShapeDtypeStruct((S, D), jnp.float32),
    )(sums2, counts, W, b.reshape(1, D))
    return out
